# baseline (device time: 39613 ns/iter reference)
import jax
import jax.numpy as jnp
from jax import lax
from jax.experimental import pallas as pl
from jax.experimental.pallas import tpu as pltpu

N_DEV = 8
N_TOK = 512
D_IN = 256
D_OUT = 512
N_EXP = 32
E_LOC = N_EXP // N_DEV
CAP = 12
CAP_PAD = 16
CHUNK = E_LOC * CAP_PAD
TOK_PER_DEV = N_TOK // N_DEV


def kernel(x, router_W, route_idx, expert_W):
    my = lax.axis_index("i")

    e = route_idx[:, 0].astype(jnp.int32)
    onehot = (e[:, None] == jnp.arange(N_EXP, dtype=jnp.int32)[None, :]).astype(
        jnp.int32
    )
    cum = jnp.cumsum(onehot, axis=0)
    rank = jnp.take_along_axis(cum, e[:, None], axis=1)[:, 0] - 1
    keep = rank < CAP

    slot = e * CAP_PAD + rank
    token_of = (
        jnp.zeros((N_EXP * CAP_PAD,), jnp.int32)
        .at[jnp.where(keep, slot, N_EXP * CAP_PAD)]
        .set(jnp.arange(N_TOK, dtype=jnp.int32), mode="drop")
    )
    src_pos = jnp.where(keep, slot, 0).astype(jnp.int32)
    keep_i32 = keep.astype(jnp.int32)

    my_tok = lax.dynamic_slice(token_of, (my * CHUNK,), (CHUNK,))
    my_src = lax.dynamic_slice(src_pos, (my * TOK_PER_DEV,), (TOK_PER_DEV,))
    my_keep = lax.dynamic_slice(keep_i32, (my * TOK_PER_DEV,), (TOK_PER_DEV,))

    def body(x_ref, w_ref, tok_ref, src_ref, keep_ref, out_ref,
             gather_ref, xg_ref, send_sems, recv_sems):
        me = lax.axis_index("i")
        left = lax.rem(me + N_DEV - 1, N_DEV)
        right = lax.rem(me + 1, N_DEV)

        barrier_sem = pltpu.get_barrier_semaphore()
        for nbr in (left, right):
            pl.semaphore_signal(
                barrier_sem, inc=1,
                device_id=(nbr,), device_id_type=pl.DeviceIdType.MESH,
            )
        pl.semaphore_wait(barrier_sem, 2)

        def gather_row(j, _):
            t = tok_ref[j]
            xg_ref[pl.ds(j, 1), :] = x_ref[pl.ds(t, 1), :]
            return 0
        lax.fori_loop(0, CHUNK, gather_row, 0)

        base = me * CHUNK
        for el in range(E_LOC):
            acc = jnp.dot(
                xg_ref[el * CAP_PAD:(el + 1) * CAP_PAD, :],
                w_ref[el],
                preferred_element_type=jnp.float32,
            )
            gather_ref[pl.ds(base + el * CAP_PAD, CAP_PAD), :] = acc

        for h in range(N_DEV - 1):
            so = lax.rem(me - h + N_DEV, N_DEV)
            rdma = pltpu.make_async_remote_copy(
                src_ref=gather_ref.at[pl.ds(so * CHUNK, CHUNK)],
                dst_ref=gather_ref.at[pl.ds(so * CHUNK, CHUNK)],
                send_sem=send_sems.at[h],
                recv_sem=recv_sems.at[h],
                device_id=(right,),
                device_id_type=pl.DeviceIdType.MESH,
            )
            rdma.start()
            rdma.wait()

        def scatter_row(j, _):
            s = src_ref[j]
            k = keep_ref[j]
            row = gather_ref[pl.ds(s, 1), :]
            out_ref[pl.ds(j, 1), :] = jnp.where(k > 0, row, 0.0)
            return 0
        lax.fori_loop(0, TOK_PER_DEV, scatter_row, 0)

    return pl.pallas_call(
        body,
        out_shape=jax.ShapeDtypeStruct((TOK_PER_DEV, D_OUT), jnp.float32),
        in_specs=[
            pl.BlockSpec(memory_space=pltpu.VMEM),
            pl.BlockSpec(memory_space=pltpu.VMEM),
            pl.BlockSpec(memory_space=pltpu.SMEM),
            pl.BlockSpec(memory_space=pltpu.SMEM),
            pl.BlockSpec(memory_space=pltpu.SMEM),
        ],
        out_specs=pl.BlockSpec(memory_space=pltpu.VMEM),
        scratch_shapes=[
            pltpu.VMEM((N_DEV * CHUNK, D_OUT), jnp.float32),
            pltpu.VMEM((CHUNK, D_IN), jnp.float32),
            pltpu.SemaphoreType.DMA((N_DEV - 1,)),
            pltpu.SemaphoreType.DMA((N_DEV - 1,)),
        ],
        compiler_params=pltpu.CompilerParams(collective_id=0),
    )(x, expert_W, my_tok, my_src, my_keep)


# device time: 17795 ns/iter; 2.2261x vs baseline; 2.2261x over previous
import jax
import jax.numpy as jnp
from jax import lax
from jax.experimental import pallas as pl
from jax.experimental.pallas import tpu as pltpu

N_DEV = 8
N_TOK = 512
D_IN = 256
D_OUT = 512
N_EXP = 32
E_LOC = N_EXP // N_DEV
CAP = 12
CAP_PAD = 16
CHUNK = E_LOC * CAP_PAD
TOK_PER_DEV = N_TOK // N_DEV


def kernel(x, router_W, route_idx, expert_W):
    def body(x_ref, w_ref, ridx_ref, out_ref, gather_ref, slotkeep_ref,
             send_sems, recv_sems):
        me = lax.axis_index("i")

        barrier_sem = pltpu.get_barrier_semaphore()
        for k in range(1, N_DEV):
            pl.semaphore_signal(
                barrier_sem, inc=1,
                device_id=(lax.rem(me + k, N_DEV),),
                device_id_type=pl.DeviceIdType.MESH,
            )
        pl.semaphore_wait(barrier_sem, N_DEV - 1)

        e_col = ridx_ref[:, :]
        iota_exp = lax.broadcasted_iota(jnp.int32, (N_TOK, N_EXP), 1)
        oh = (e_col == iota_exp).astype(jnp.float32)
        ti = lax.broadcasted_iota(jnp.int32, (N_TOK, N_TOK), 0)
        tj = lax.broadcasted_iota(jnp.int32, (N_TOK, N_TOK), 1)
        lower = (tj <= ti).astype(jnp.float32)
        cum = jnp.dot(lower, oh, preferred_element_type=jnp.float32)
        rank = jnp.sum(cum * oh, axis=1, keepdims=True) - 1.0
        keep = rank < float(CAP)
        slot = e_col * CAP_PAD + rank.astype(jnp.int32)
        slotkeep_ref[:, 0:1] = slot.astype(jnp.float32)
        slotkeep_ref[:, 1:2] = jnp.where(keep, 1.0, 0.0)

        iota_ch = lax.broadcasted_iota(jnp.int32, (N_TOK, CHUNK), 1)
        osl = jnp.where(
            keep & ((slot - me * CHUNK) == iota_ch), 1.0, 0.0
        )
        xg = lax.dot_general(
            osl, x_ref[:, :],
            dimension_numbers=(((0,), (0,)), ((), ())),
            preferred_element_type=jnp.float32,
        )
        chunk = jnp.concatenate(
            [
                jnp.dot(
                    xg[el * CAP_PAD:(el + 1) * CAP_PAD, :],
                    w_ref[el],
                    preferred_element_type=jnp.float32,
                )
                for el in range(E_LOC)
            ],
            axis=0,
        )
        gather_ref[pl.ds(me * CHUNK, CHUNK), :] = chunk

        sends = []
        for k in range(1, N_DEV):
            rdma = pltpu.make_async_remote_copy(
                src_ref=gather_ref.at[pl.ds(me * CHUNK, CHUNK)],
                dst_ref=gather_ref.at[pl.ds(me * CHUNK, CHUNK)],
                send_sem=send_sems.at[k - 1],
                recv_sem=recv_sems.at[me],
                device_id=(lax.rem(me + k, N_DEV),),
                device_id_type=pl.DeviceIdType.MESH,
            )
            rdma.start()
            sends.append(rdma)

        mine = slotkeep_ref[pl.ds(me * TOK_PER_DEV, TOK_PER_DEV), :]
        slot_my = mine[:, 0:1].astype(jnp.int32)
        keep_my = mine[:, 1:2] > 0.5
        iota_all = lax.broadcasted_iota(
            jnp.int32, (TOK_PER_DEV, N_DEV * CHUNK), 1
        )
        scat = jnp.where(keep_my & (slot_my == iota_all), 1.0, 0.0)

        for k in range(1, N_DEV):
            o = lax.rem(me + N_DEV - k, N_DEV)
            recv = pltpu.make_async_remote_copy(
                src_ref=gather_ref.at[pl.ds(o * CHUNK, CHUNK)],
                dst_ref=gather_ref.at[pl.ds(o * CHUNK, CHUNK)],
                send_sem=send_sems.at[k - 1],
                recv_sem=recv_sems.at[o],
                device_id=(me,),
                device_id_type=pl.DeviceIdType.MESH,
            )
            recv.wait_recv()
        for rdma in sends:
            rdma.wait_send()

        out_ref[:, :] = jnp.dot(
            scat, gather_ref[:, :], preferred_element_type=jnp.float32
        )

    return pl.pallas_call(
        body,
        out_shape=jax.ShapeDtypeStruct((TOK_PER_DEV, D_OUT), jnp.float32),
        in_specs=[
            pl.BlockSpec(memory_space=pltpu.VMEM),
            pl.BlockSpec(memory_space=pltpu.VMEM),
            pl.BlockSpec(memory_space=pltpu.VMEM),
        ],
        out_specs=pl.BlockSpec(memory_space=pltpu.VMEM),
        scratch_shapes=[
            pltpu.VMEM((N_DEV * CHUNK, D_OUT), jnp.float32),
            pltpu.VMEM((N_TOK, 2), jnp.float32),
            pltpu.SemaphoreType.DMA((N_DEV - 1,)),
            pltpu.SemaphoreType.DMA((N_DEV,)),
        ],
        compiler_params=pltpu.CompilerParams(collective_id=0),
    )(x, expert_W, route_idx)


# device time: 14971 ns/iter; 2.6460x vs baseline; 1.1886x over previous
import jax
import jax.numpy as jnp
from jax import lax
from jax.experimental import pallas as pl
from jax.experimental.pallas import tpu as pltpu

N_DEV = 8
N_TOK = 512
D_IN = 256
D_OUT = 512
N_EXP = 32
E_LOC = N_EXP // N_DEV
CAP = 12
CAP_PAD = 12
CHUNK = E_LOC * CAP_PAD
TOK_PER_DEV = N_TOK // N_DEV


def kernel(x, router_W, route_idx, expert_W):
    def body(x_ref, w_ref, ridx_ref, out_ref, gather_ref, slotkeep_ref,
             send_sems, recv_sems):
        me = lax.axis_index("i")

        barrier_sem = pltpu.get_barrier_semaphore()
        for k in range(1, N_DEV):
            pl.semaphore_signal(
                barrier_sem, inc=1,
                device_id=(lax.rem(me + k, N_DEV),),
                device_id_type=pl.DeviceIdType.MESH,
            )

        e_col = ridx_ref[:, :]
        iota_exp = lax.broadcasted_iota(jnp.int32, (N_TOK, N_EXP), 1)
        oh = (e_col == iota_exp).astype(jnp.float32)
        ti = lax.broadcasted_iota(jnp.int32, (N_TOK, N_TOK), 0)
        tj = lax.broadcasted_iota(jnp.int32, (N_TOK, N_TOK), 1)
        lower = (tj <= ti).astype(jnp.float32)
        cum = jnp.dot(lower, oh, preferred_element_type=jnp.float32)
        rank = jnp.sum(cum * oh, axis=1, keepdims=True) - 1.0
        keep = rank < float(CAP)
        slot = e_col * CAP_PAD + rank.astype(jnp.int32)
        slotkeep_ref[:, 0:1] = slot.astype(jnp.float32)
        slotkeep_ref[:, 1:2] = jnp.where(keep, 1.0, 0.0)

        iota_ch = lax.broadcasted_iota(jnp.int32, (N_TOK, CHUNK), 1)
        osl = jnp.where(
            keep & ((slot - me * CHUNK) == iota_ch), 1.0, 0.0
        )
        xg = lax.dot_general(
            osl, x_ref[:, :],
            dimension_numbers=(((0,), (0,)), ((), ())),
            preferred_element_type=jnp.float32,
        )
        chunk = jnp.concatenate(
            [
                jnp.dot(
                    xg[el * CAP_PAD:(el + 1) * CAP_PAD, :],
                    w_ref[el],
                    preferred_element_type=jnp.float32,
                )
                for el in range(E_LOC)
            ],
            axis=0,
        )
        gather_ref[pl.ds(me * CHUNK, CHUNK), :] = chunk

        pl.semaphore_wait(barrier_sem, N_DEV - 1)

        sends = []
        for k in range(1, N_DEV):
            rdma = pltpu.make_async_remote_copy(
                src_ref=gather_ref.at[pl.ds(me * CHUNK, CHUNK)],
                dst_ref=gather_ref.at[pl.ds(me * CHUNK, CHUNK)],
                send_sem=send_sems.at[k - 1],
                recv_sem=recv_sems.at[me],
                device_id=(lax.rem(me + k, N_DEV),),
                device_id_type=pl.DeviceIdType.MESH,
            )
            rdma.start()
            sends.append(rdma)

        mine = slotkeep_ref[pl.ds(me * TOK_PER_DEV, TOK_PER_DEV), :]
        slot_my = mine[:, 0:1].astype(jnp.int32)
        keep_my = mine[:, 1:2] > 0.5
        iota_all = lax.broadcasted_iota(
            jnp.int32, (TOK_PER_DEV, N_DEV * CHUNK), 1
        )
        scat = jnp.where(keep_my & (slot_my == iota_all), 1.0, 0.0)

        for k in range(1, N_DEV):
            o = lax.rem(me + N_DEV - k, N_DEV)
            recv = pltpu.make_async_remote_copy(
                src_ref=gather_ref.at[pl.ds(o * CHUNK, CHUNK)],
                dst_ref=gather_ref.at[pl.ds(o * CHUNK, CHUNK)],
                send_sem=send_sems.at[k - 1],
                recv_sem=recv_sems.at[o],
                device_id=(me,),
                device_id_type=pl.DeviceIdType.MESH,
            )
            recv.wait_recv()
        for rdma in sends:
            rdma.wait_send()

        out_ref[:, :] = jnp.dot(
            scat, gather_ref[:, :], preferred_element_type=jnp.float32
        )

    return pl.pallas_call(
        body,
        out_shape=jax.ShapeDtypeStruct((TOK_PER_DEV, D_OUT), jnp.float32),
        in_specs=[
            pl.BlockSpec(memory_space=pltpu.VMEM),
            pl.BlockSpec(memory_space=pltpu.VMEM),
            pl.BlockSpec(memory_space=pltpu.VMEM),
        ],
        out_specs=pl.BlockSpec(memory_space=pltpu.VMEM),
        scratch_shapes=[
            pltpu.VMEM((N_DEV * CHUNK, D_OUT), jnp.float32),
            pltpu.VMEM((N_TOK, 2), jnp.float32),
            pltpu.SemaphoreType.DMA((N_DEV - 1,)),
            pltpu.SemaphoreType.DMA((N_DEV,)),
        ],
        compiler_params=pltpu.CompilerParams(collective_id=0),
    )(x, expert_W, route_idx)


# device time: 12923 ns/iter; 3.0653x vs baseline; 1.1585x over previous
import jax
import jax.numpy as jnp
from jax import lax
from jax.experimental import pallas as pl
from jax.experimental.pallas import tpu as pltpu

N_DEV = 8
N_TOK = 512
D_IN = 256
D_OUT = 512
N_EXP = 32
E_LOC = N_EXP // N_DEV
CAP = 12
CAP_PAD = 12
CHUNK = E_LOC * CAP_PAD
TOK_PER_DEV = N_TOK // N_DEV


def kernel(x, router_W, route_idx, expert_W):
    def body(x_ref, w_ref, ridx_ref, out_ref, gather_ref, slotkeep_ref,
             send_sems, recv_sems):
        me = lax.axis_index("i")

        barrier_sem = pltpu.get_barrier_semaphore()
        for k in range(1, N_DEV):
            pl.semaphore_signal(
                barrier_sem, inc=1,
                device_id=(lax.rem(me + k, N_DEV),),
                device_id_type=pl.DeviceIdType.MESH,
            )

        e_col = ridx_ref[:, :]
        iota_exp = lax.broadcasted_iota(jnp.int32, (N_TOK, N_EXP), 1)
        oh = (e_col == iota_exp).astype(jnp.float32)
        ti = lax.broadcasted_iota(jnp.int32, (N_TOK, N_TOK), 0)
        tj = lax.broadcasted_iota(jnp.int32, (N_TOK, N_TOK), 1)
        lower = (tj <= ti).astype(jnp.float32)
        cum = jnp.dot(lower, oh, preferred_element_type=jnp.float32)
        rank = jnp.sum(cum * oh, axis=1, keepdims=True) - 1.0
        keep = rank < float(CAP)
        slot = e_col * CAP_PAD + rank.astype(jnp.int32)
        slotkeep_ref[:, 0:1] = slot.astype(jnp.float32)
        slotkeep_ref[:, 1:2] = jnp.where(keep, 1.0, 0.0)

        iota_ch = lax.broadcasted_iota(jnp.int32, (N_TOK, CHUNK), 1)
        osl = jnp.where(
            keep & ((slot - me * CHUNK) == iota_ch), 1.0, 0.0
        )
        xg = lax.dot_general(
            osl, x_ref[:, :],
            dimension_numbers=(((0,), (0,)), ((), ())),
            preferred_element_type=jnp.float32,
        )
        chunk = jnp.concatenate(
            [
                jnp.dot(
                    xg[el * CAP_PAD:(el + 1) * CAP_PAD, :],
                    w_ref[el],
                    preferred_element_type=jnp.float32,
                )
                for el in range(E_LOC)
            ],
            axis=0,
        )
        gather_ref[pl.ds(me * CHUNK, CHUNK), :] = chunk.astype(jnp.bfloat16)

        pl.semaphore_wait(barrier_sem, N_DEV - 1)

        sends = []
        for k in range(1, N_DEV):
            rdma = pltpu.make_async_remote_copy(
                src_ref=gather_ref.at[pl.ds(me * CHUNK, CHUNK)],
                dst_ref=gather_ref.at[pl.ds(me * CHUNK, CHUNK)],
                send_sem=send_sems.at[k - 1],
                recv_sem=recv_sems.at[me],
                device_id=(lax.rem(me + k, N_DEV),),
                device_id_type=pl.DeviceIdType.MESH,
            )
            rdma.start()
            sends.append(rdma)

        mine = slotkeep_ref[pl.ds(me * TOK_PER_DEV, TOK_PER_DEV), :]
        slot_my = mine[:, 0:1].astype(jnp.int32)
        keep_my = mine[:, 1:2] > 0.5
        iota_all = lax.broadcasted_iota(
            jnp.int32, (TOK_PER_DEV, N_DEV * CHUNK), 1
        )
        scat = jnp.where(
            keep_my & (slot_my == iota_all), 1.0, 0.0
        ).astype(jnp.bfloat16)

        for k in range(1, N_DEV):
            o = lax.rem(me + N_DEV - k, N_DEV)
            recv = pltpu.make_async_remote_copy(
                src_ref=gather_ref.at[pl.ds(o * CHUNK, CHUNK)],
                dst_ref=gather_ref.at[pl.ds(o * CHUNK, CHUNK)],
                send_sem=send_sems.at[k - 1],
                recv_sem=recv_sems.at[o],
                device_id=(me,),
                device_id_type=pl.DeviceIdType.MESH,
            )
            recv.wait_recv()
        for rdma in sends:
            rdma.wait_send()

        out_ref[:, :] = jnp.dot(
            scat, gather_ref[:, :], preferred_element_type=jnp.float32
        )

    return pl.pallas_call(
        body,
        out_shape=jax.ShapeDtypeStruct((TOK_PER_DEV, D_OUT), jnp.float32),
        in_specs=[
            pl.BlockSpec(memory_space=pltpu.VMEM),
            pl.BlockSpec(memory_space=pltpu.VMEM),
            pl.BlockSpec(memory_space=pltpu.VMEM),
        ],
        out_specs=pl.BlockSpec(memory_space=pltpu.VMEM),
        scratch_shapes=[
            pltpu.VMEM((N_DEV * CHUNK, D_OUT), jnp.bfloat16),
            pltpu.VMEM((N_TOK, 2), jnp.float32),
            pltpu.SemaphoreType.DMA((N_DEV - 1,)),
            pltpu.SemaphoreType.DMA((N_DEV,)),
        ],
        compiler_params=pltpu.CompilerParams(collective_id=0),
    )(x, expert_W, route_idx)
